# baseline (speedup 1.0000x reference)
import jax
import jax.numpy as jnp
from jax import lax
from jax.experimental import pallas as pl
from jax.experimental.pallas import tpu as pltpu
from jax.experimental.pallas import tpu_sc as plsc

BATCH = 16384
NF = 26
D = 16
VOCAB = 100000
XW = 2 * NF
OUT_W = NF * D + NF
BLK = 128
VPB = BLK // D
NBLK = VOCAB // VPB

NC = 2
NS = 16
NW = NC * NS
B_PER_W = BATCH // NW
BC = 128
N_CHUNK = B_PER_W // BC
L = 16


def _body(x_hbm, tab_hbm, out_hbm, xv, blk_v, low_v, bbuf, stage_v, gsem, osem):
    wid = lax.axis_index("s") * NC + lax.axis_index("c")
    iota = lax.iota(jnp.int32, L)

    def chunk(c, carry):
        base = wid * B_PER_W + c * BC

        pltpu.sync_copy(x_hbm.at[pl.ds(base, BC), :], xv)

        for j in range(NF * BC // L):
            f = j // (BC // L)
            b0 = (j % (BC // L)) * L
            vals = plsc.load_gather(
                xv, [b0 + iota, jnp.full((L,), f, jnp.int32)]
            )
            blk_v[f, pl.ds(b0, L)] = vals >> 3
            low_v[f, pl.ds(b0, L)] = (vals & 7) << 4

        for j in range(NF * BC // L):
            col = j // (BC // L)
            b0 = (j % (BC // L)) * L
            vals = plsc.load_gather(
                xv, [b0 + iota, jnp.full((L,), NF + col, jnp.int32)]
            )
            stage_v[NF * D + col, pl.ds(b0, L)] = vals.astype(jnp.float32)

        pltpu.async_copy(
            tab_hbm.at[0].at[blk_v.at[0]], bbuf.at[pl.ds(0, BC), :], gsem
        )

        def feat(f, carry2):
            nxt = f + 1

            @pl.when(nxt < NF)
            def _():
                pltpu.async_copy(
                    tab_hbm.at[nxt].at[blk_v.at[nxt]],
                    bbuf.at[pl.ds((nxt % 2) * BC, BC), :],
                    gsem,
                )

            pltpu.make_async_copy(
                tab_hbm.at[0].at[blk_v.at[0]],
                bbuf.at[pl.ds((f % 2) * BC, BC), :],
                gsem,
            ).wait()

            bufbase = (f % 2) * BC
            for b0 in range(BC // L):
                rows = bufbase + b0 * L + iota
                lowvec = low_v[f, pl.ds(b0 * L, L)]
                for d in range(D):
                    vals = plsc.load_gather(bbuf, [rows, lowvec + d])
                    stage_v[f * D + d, pl.ds(b0 * L, L)] = vals
            return carry2

        lax.fori_loop(0, NF, feat, 0)

        pltpu.async_copy(stage_v, out_hbm.at[:, pl.ds(base, BC)], osem).wait()
        return carry

    lax.fori_loop(0, N_CHUNK, chunk, 0)


@jax.jit
def _emb_lookup(x, tab5):
    run = pl.kernel(
        _body,
        out_type=jax.ShapeDtypeStruct((OUT_W, BATCH), jnp.float32),
        mesh=plsc.VectorSubcoreMesh(
            core_axis_name="c", subcore_axis_name="s", num_cores=NC,
            num_subcores=NS,
        ),
        scratch_types=[
            pltpu.VMEM((BC, XW), jnp.int32),
            pltpu.VMEM((NF, BC), jnp.int32),
            pltpu.VMEM((NF, BC), jnp.int32),
            pltpu.VMEM((2 * BC, BLK), jnp.float32),
            pltpu.VMEM((OUT_W, BC), jnp.float32),
            pltpu.SemaphoreType.DMA,
            pltpu.SemaphoreType.DMA,
        ],
        compiler_params=pltpu.CompilerParams(
            use_tc_tiling_on_sc=False, needs_layout_passes=False
        ),
    )
    return run(x, tab5)


def kernel(x, tables):
    return _emb_lookup(x, tables.reshape(NF, NBLK, BLK)).T
